# baseline (speedup 1.0000x reference)
import functools

import jax
import jax.numpy as jnp
from jax import lax
from jax.experimental import pallas as pl
from jax.experimental.pallas import tpu as pltpu
from jax.experimental.pallas import tpu_sc as plsc

_N = 10000
_E = 320000
_D = 128

_NC = 2
_NS = 16
_NW = _NC * _NS

_CHUNK = 128
_G = 4
_CPG = 20
_NCHUNK = _G * _CPG
_EPT = _NCHUNK * _CHUNK
_EPAD = _NW * _EPT
_NPAD = 10240
_RPT = _NPAD // _NS


def _sc_aggregate_kernel(h_hbm, src_hbm, dst_hbm, z_hbm, out_hbm,
                         agg, src_blk, dst_blk, rows0, rows1,
                         sem0, sem1, ssem0, ssem1, sem_isrc, sem_idst):
    c = lax.axis_index("c")
    s = lax.axis_index("s")
    wid = c * _NS + s

    pltpu.sync_copy(z_hbm.at[pl.ds(s * _RPT, _RPT)],
                    agg.at[pl.ds(s * _RPT, _RPT)])
    plsc.subcore_barrier()

    bufs = (rows0, rows1)
    sems = (sem0, sem1)
    ssems = (ssem0, ssem1)

    def start_idx(g):
        slot = g % 2
        pltpu.async_copy(src_hbm.at[wid, g], src_blk.at[slot], sem_isrc)
        pltpu.async_copy(dst_hbm.at[wid, g], dst_blk.at[slot], sem_idst)

    def wait_idx(g):
        slot = g % 2
        pltpu.make_async_copy(src_hbm.at[wid, g], src_blk.at[slot],
                              sem_isrc).wait()
        pltpu.make_async_copy(dst_hbm.at[wid, g], dst_blk.at[slot],
                              sem_idst).wait()

    def start_gather(slot, j, b):
        pltpu.async_copy(h_hbm.at[src_blk.at[slot, j]], bufs[b], sems[b])

    def wait_gather(slot, j, b):
        pltpu.make_async_copy(h_hbm.at[src_blk.at[slot, j]], bufs[b],
                              sems[b]).wait()

    def start_scatter(slot, j, b):
        pltpu.async_copy(bufs[b], agg.at[dst_blk.at[slot, j]], ssems[b],
                         add=True)

    def wait_scatter(slot, j, b):
        pltpu.make_async_copy(bufs[b], agg.at[dst_blk.at[slot, j]],
                              ssems[b]).wait()

    start_idx(0)
    for g in range(_G):
        slot = g % 2
        wait_idx(g)
        if g + 1 < _G:
            start_idx(g + 1)
        start_gather(slot, 0, 0)

        def body(jj, carry):
            for b in range(2):
                j = 2 * jj + b
                wait_gather(slot, j, b)
                start_scatter(slot, j, b)

                @pl.when(j + 1 < _CPG)
                def _():
                    if b == 1:
                        wait_scatter(slot, j - 1, 0)
                        start_gather(slot, j + 1, 0)
                    else:
                        @pl.when(j >= 1)
                        def _():
                            wait_scatter(slot, j - 1, 1)

                        start_gather(slot, j + 1, 1)
            return carry

        lax.fori_loop(0, _CPG // 2, body, 0)
        wait_scatter(slot, _CPG - 2, 0)
        wait_scatter(slot, _CPG - 1, 1)

    plsc.subcore_barrier()

    pltpu.sync_copy(agg.at[pl.ds(s * _RPT, _RPT)],
                    out_hbm.at[c, pl.ds(s * _RPT, _RPT)])


def _sc_aggregate(h_pad, src4, dst4, zeros_pad):
    mesh = plsc.VectorSubcoreMesh(core_axis_name="c", subcore_axis_name="s")
    return pl.kernel(
        _sc_aggregate_kernel,
        out_type=jax.ShapeDtypeStruct((_NC, _NPAD, _D), jnp.float32),
        mesh=mesh,
        scratch_types=[
            pltpu.VMEM_SHARED((_NPAD, _D), jnp.float32),
            pltpu.VMEM((2, _CPG, _CHUNK), jnp.int32),
            pltpu.VMEM((2, _CPG, _CHUNK), jnp.int32),
            pltpu.VMEM((_CHUNK, _D), jnp.float32),
            pltpu.VMEM((_CHUNK, _D), jnp.float32),
            pltpu.SemaphoreType.DMA,
            pltpu.SemaphoreType.DMA,
            pltpu.SemaphoreType.DMA,
            pltpu.SemaphoreType.DMA,
            pltpu.SemaphoreType.DMA,
            pltpu.SemaphoreType.DMA,
        ],
    )(h_pad, src4, dst4, zeros_pad)


def _tc_layer_kernel(do_relu, p_ref, h_ref, wr_ref, b_ref, wo_ref, o_ref):
    agg = p_ref[0] + p_ref[1]
    acc = jnp.dot(agg, wr_ref[...], preferred_element_type=jnp.float32)
    acc = acc + jnp.dot(h_ref[...], wo_ref[...],
                        preferred_element_type=jnp.float32)
    acc = acc + b_ref[...]
    if do_relu:
        acc = jnp.maximum(acc, 0.0)
    o_ref[...] = acc


def _tc_layer(partials, h_pad, w_rel_t, b_rel, w_root_t, do_relu):
    blk = 1024
    grid = (_NPAD // blk,)
    return pl.pallas_call(
        functools.partial(_tc_layer_kernel, do_relu),
        grid=grid,
        in_specs=[
            pl.BlockSpec((_NC, blk, _D), lambda i: (0, i, 0)),
            pl.BlockSpec((blk, _D), lambda i: (i, 0)),
            pl.BlockSpec((_D, _D), lambda i: (0, 0)),
            pl.BlockSpec((1, _D), lambda i: (0, 0)),
            pl.BlockSpec((_D, _D), lambda i: (0, 0)),
        ],
        out_specs=pl.BlockSpec((blk, _D), lambda i: (i, 0)),
        out_shape=jax.ShapeDtypeStruct((_NPAD, _D), jnp.float32),
    )(partials, h_pad, w_rel_t, b_rel, w_root_t)


def kernel(x, edge_index, W1_rel, b1_rel, W1_root, W2_rel, b2_rel, W2_root,
           W3_rel, b3_rel, W3_root):
    src = edge_index[0]
    dst = edge_index[1]

    pad_n = _EPAD - _E
    pad_idx = (_N + (jnp.arange(pad_n, dtype=jnp.int32) % (_NPAD - _N)))
    src4 = jnp.concatenate([src, pad_idx]).reshape(_NW, _G, _CPG, _CHUNK)
    dst4 = jnp.concatenate([dst, pad_idx]).reshape(_NW, _G, _CPG, _CHUNK)

    h = jnp.pad(x, ((0, _NPAD - _N), (0, 0)))
    zeros_pad = jnp.zeros((_NPAD, _D), jnp.float32)

    layers = (
        (W1_rel, b1_rel, W1_root, True),
        (W2_rel, b2_rel, W2_root, True),
        (W3_rel, b3_rel, W3_root, False),
    )
    for w_rel, b_rel, w_root, do_relu in layers:
        partials = _sc_aggregate(h, src4, dst4, zeros_pad)
        h = _tc_layer(partials, h, w_rel.T, b_rel.reshape(1, _D),
                      w_root.T, do_relu)
    return h[:_N]


# speedup vs baseline: 1.1042x; 1.1042x over previous
import functools

import jax
import jax.numpy as jnp
from jax import lax
from jax.experimental import pallas as pl
from jax.experimental.pallas import tpu as pltpu
from jax.experimental.pallas import tpu_sc as plsc

_N = 10000
_E = 320000
_D = 128

_NC = 2
_NS = 16
_NW = _NC * _NS

_CHUNK = 100
_G = 5
_CPG = 20
_NCHUNK = _G * _CPG
_EPT = _NCHUNK * _CHUNK
_NPAD = 10112
_RPT = _NPAD // _NS


def _sc_aggregate_kernel(h_hbm, edge_hbm, z_hbm, out_hbm,
                         agg, src_blk, dst_blk, rows0, rows1,
                         sem0, sem1, sem_isrc, sem_idst):
    c = lax.axis_index("c")
    s = lax.axis_index("s")
    wid = c * _NS + s

    def start_idx(g):
        slot = g % 2
        pltpu.async_copy(edge_hbm.at[0, wid, g], src_blk.at[slot], sem_isrc)
        pltpu.async_copy(edge_hbm.at[1, wid, g], dst_blk.at[slot], sem_idst)

    def wait_idx(g):
        slot = g % 2
        pltpu.make_async_copy(edge_hbm.at[0, wid, g], src_blk.at[slot],
                              sem_isrc).wait()
        pltpu.make_async_copy(edge_hbm.at[1, wid, g], dst_blk.at[slot],
                              sem_idst).wait()

    start_idx(0)

    pltpu.sync_copy(z_hbm.at[pl.ds(s * _RPT, _RPT)],
                    agg.at[pl.ds(s * _RPT, _RPT)])
    plsc.subcore_barrier()

    bufs = (rows0, rows1)
    sems = (sem0, sem1)

    def start_gather(slot, j, b):
        pltpu.async_copy(h_hbm.at[src_blk.at[slot, j]], bufs[b], sems[b])

    def wait_gather(slot, j, b):
        pltpu.make_async_copy(h_hbm.at[src_blk.at[slot, j]], bufs[b],
                              sems[b]).wait()

    for g in range(_G):
        slot = g % 2
        wait_idx(g)
        if g + 1 < _G:
            start_idx(g + 1)
        start_gather(slot, 0, 0)
        start_gather(slot, 1, 1)

        def body(jj, carry):
            for b in range(2):
                j = 2 * jj + b
                wait_gather(slot, j, b)
                pltpu.sync_copy(bufs[b], agg.at[dst_blk.at[slot, j]],
                                add=True)

                @pl.when(j + 2 < _CPG)
                def _():
                    start_gather(slot, j + 2, b)
            return carry

        lax.fori_loop(0, _CPG // 2, body, 0)

    plsc.subcore_barrier()

    pltpu.sync_copy(agg.at[pl.ds(s * _RPT, _RPT)],
                    out_hbm.at[c, pl.ds(s * _RPT, _RPT)])


def _sc_aggregate(h, edge5, zeros_n):
    mesh = plsc.VectorSubcoreMesh(core_axis_name="c", subcore_axis_name="s")
    return pl.kernel(
        _sc_aggregate_kernel,
        out_type=jax.ShapeDtypeStruct((_NC, _NPAD, _D), jnp.float32),
        mesh=mesh,
        scratch_types=[
            pltpu.VMEM_SHARED((_NPAD, _D), jnp.float32),
            pltpu.VMEM((2, _CPG, _CHUNK), jnp.int32),
            pltpu.VMEM((2, _CPG, _CHUNK), jnp.int32),
            pltpu.VMEM((_CHUNK, _D), jnp.float32),
            pltpu.VMEM((_CHUNK, _D), jnp.float32),
            pltpu.SemaphoreType.DMA,
            pltpu.SemaphoreType.DMA,
            pltpu.SemaphoreType.DMA,
            pltpu.SemaphoreType.DMA,
        ],
    )(h, edge5, zeros_n)


def _tc_layer_kernel(do_relu, p_ref, h_ref, wr_ref, b_ref, wo_ref, o_ref):
    agg = p_ref[0] + p_ref[1]
    acc = jnp.dot(agg, wr_ref[...], preferred_element_type=jnp.float32)
    acc = acc + jnp.dot(h_ref[...], wo_ref[...],
                        preferred_element_type=jnp.float32)
    acc = acc + b_ref[...]
    if do_relu:
        acc = jnp.maximum(acc, 0.0)
    o_ref[...] = acc


def _tc_layer(partials, h, w_rel_t, b_rel, w_root_t, do_relu):
    blk = 400
    grid = (_N // blk,)
    return pl.pallas_call(
        functools.partial(_tc_layer_kernel, do_relu),
        grid=grid,
        in_specs=[
            pl.BlockSpec((_NC, blk, _D), lambda i: (0, i, 0)),
            pl.BlockSpec((blk, _D), lambda i: (i, 0)),
            pl.BlockSpec((_D, _D), lambda i: (0, 0)),
            pl.BlockSpec((1, _D), lambda i: (0, 0)),
            pl.BlockSpec((_D, _D), lambda i: (0, 0)),
        ],
        out_specs=pl.BlockSpec((blk, _D), lambda i: (i, 0)),
        out_shape=jax.ShapeDtypeStruct((_N, _D), jnp.float32),
    )(partials, h, w_rel_t, b_rel, w_root_t)


def kernel(x, edge_index, W1_rel, b1_rel, W1_root, W2_rel, b2_rel, W2_root,
           W3_rel, b3_rel, W3_root):
    edge5 = edge_index.reshape(2, _NW, _G, _CPG, _CHUNK)
    zeros_n = jnp.zeros((_NPAD, _D), jnp.float32)

    h = x
    layers = (
        (W1_rel, b1_rel, W1_root, True),
        (W2_rel, b2_rel, W2_root, True),
        (W3_rel, b3_rel, W3_root, False),
    )
    for w_rel, b_rel, w_root, do_relu in layers:
        partials = _sc_aggregate(h, edge5, zeros_n)
        h = _tc_layer(partials, h, w_rel.T, b_rel.reshape(1, _D),
                      w_root.T, do_relu)
    return h


# speedup vs baseline: 1.1920x; 1.0796x over previous
import functools

import jax
import jax.numpy as jnp
from jax import lax
from jax.experimental import pallas as pl
from jax.experimental.pallas import tpu as pltpu
from jax.experimental.pallas import tpu_sc as plsc

_N = 10000
_E = 320000
_D = 128

_NC = 2
_NS = 16
_NW = _NC * _NS

_CHUNK = 100
_G = 5
_CPG = 20
_NCHUNK = _G * _CPG
_EPT = _NCHUNK * _CHUNK
_NPAD = 10112
_RPT = _NPAD // _NS


def _sc_aggregate_kernel(h_hbm, edge_hbm, z_hbm, out_hbm,
                         agg, src_blk, dst_blk, rows0, rows1,
                         sem0, sem1, sem_isrc, sem_idst):
    c = lax.axis_index("c")
    s = lax.axis_index("s")
    wid = c * _NS + s

    def start_idx(g):
        slot = g % 2
        pltpu.async_copy(edge_hbm.at[0, wid, g], src_blk.at[slot], sem_isrc)
        pltpu.async_copy(edge_hbm.at[1, wid, g], dst_blk.at[slot], sem_idst)

    def wait_idx(g):
        slot = g % 2
        pltpu.make_async_copy(edge_hbm.at[0, wid, g], src_blk.at[slot],
                              sem_isrc).wait()
        pltpu.make_async_copy(edge_hbm.at[1, wid, g], dst_blk.at[slot],
                              sem_idst).wait()

    start_idx(0)

    pltpu.sync_copy(z_hbm.at[pl.ds(s * _RPT, _RPT)],
                    agg.at[pl.ds(s * _RPT, _RPT)])
    plsc.subcore_barrier()

    bufs = (rows0, rows1)
    sems = (sem0, sem1)

    def start_gather(slot, j, b):
        pltpu.async_copy(h_hbm.at[src_blk.at[slot, j]], bufs[b], sems[b])

    def wait_gather(slot, j, b):
        pltpu.make_async_copy(h_hbm.at[src_blk.at[slot, j]], bufs[b],
                              sems[b]).wait()

    for g in range(_G):
        slot = g % 2
        wait_idx(g)
        if g + 1 < _G:
            start_idx(g + 1)
        start_gather(slot, 0, 0)
        start_gather(slot, 1, 1)

        def body(jj, carry):
            for b in range(2):
                j = 2 * jj + b
                wait_gather(slot, j, b)
                pltpu.sync_copy(bufs[b], agg.at[dst_blk.at[slot, j]],
                                add=True)

                @pl.when(j + 2 < _CPG)
                def _():
                    start_gather(slot, j + 2, b)
            return carry

        lax.fori_loop(0, _CPG // 2, body, 0)

    plsc.subcore_barrier()

    pltpu.sync_copy(agg.at[pl.ds(s * _RPT, _RPT)],
                    out_hbm.at[c, pl.ds(s * _RPT, _RPT)])


def _sc_aggregate(h, edge5, zeros_n):
    mesh = plsc.VectorSubcoreMesh(core_axis_name="c", subcore_axis_name="s")
    return pl.kernel(
        _sc_aggregate_kernel,
        out_type=jax.ShapeDtypeStruct((_NC, _NPAD, _D), jnp.float32),
        mesh=mesh,
        scratch_types=[
            pltpu.VMEM_SHARED((_NPAD, _D), jnp.float32),
            pltpu.VMEM((2, _CPG, _CHUNK), jnp.int32),
            pltpu.VMEM((2, _CPG, _CHUNK), jnp.int32),
            pltpu.VMEM((_CHUNK, _D), jnp.float32),
            pltpu.VMEM((_CHUNK, _D), jnp.float32),
            pltpu.SemaphoreType.DMA,
            pltpu.SemaphoreType.DMA,
            pltpu.SemaphoreType.DMA,
            pltpu.SemaphoreType.DMA,
        ],
    )(h, edge5, zeros_n)


def _tc_root_kernel(h_ref, wo_ref, b_ref, o_ref):
    o_ref[...] = jnp.dot(h_ref[...], wo_ref[...],
                         preferred_element_type=jnp.float32) + b_ref[...]


def _tc_root(h, w_root_t, b_rel):
    blk = 2000
    return pl.pallas_call(
        _tc_root_kernel,
        grid=(_N // blk,),
        in_specs=[
            pl.BlockSpec((blk, _D), lambda i: (i, 0)),
            pl.BlockSpec((_D, _D), lambda i: (0, 0)),
            pl.BlockSpec((1, _D), lambda i: (0, 0)),
        ],
        out_specs=pl.BlockSpec((blk, _D), lambda i: (i, 0)),
        out_shape=jax.ShapeDtypeStruct((_N, _D), jnp.float32),
    )(h, w_root_t, b_rel)


def _tc_combine_kernel(do_relu, p_ref, r_ref, wr_ref, o_ref):
    agg = p_ref[0] + p_ref[1]
    acc = jnp.dot(agg, wr_ref[...], preferred_element_type=jnp.float32)
    acc = acc + r_ref[...]
    if do_relu:
        acc = jnp.maximum(acc, 0.0)
    o_ref[...] = acc


def _tc_combine(partials, root, w_rel_t, do_relu):
    blk = 2000
    return pl.pallas_call(
        functools.partial(_tc_combine_kernel, do_relu),
        grid=(_N // blk,),
        in_specs=[
            pl.BlockSpec((_NC, blk, _D), lambda i: (0, i, 0)),
            pl.BlockSpec((blk, _D), lambda i: (i, 0)),
            pl.BlockSpec((_D, _D), lambda i: (0, 0)),
        ],
        out_specs=pl.BlockSpec((blk, _D), lambda i: (i, 0)),
        out_shape=jax.ShapeDtypeStruct((_N, _D), jnp.float32),
    )(partials, root, w_rel_t)


def kernel(x, edge_index, W1_rel, b1_rel, W1_root, W2_rel, b2_rel, W2_root,
           W3_rel, b3_rel, W3_root):
    edge5 = edge_index.reshape(2, _NW, _G, _CPG, _CHUNK)
    zeros_n = jnp.zeros((_NPAD, _D), jnp.float32)

    h = x
    layers = (
        (W1_rel, b1_rel, W1_root, True),
        (W2_rel, b2_rel, W2_root, True),
        (W3_rel, b3_rel, W3_root, False),
    )
    for w_rel, b_rel, w_root, do_relu in layers:
        partials = _sc_aggregate(h, edge5, zeros_n)
        root = _tc_root(h, w_root.T, b_rel.reshape(1, _D))
        h = _tc_combine(partials, root, w_rel.T, do_relu)
    return h


# speedup vs baseline: 1.2200x; 1.0235x over previous
import functools

import jax
import jax.numpy as jnp
from jax import lax
from jax.experimental import pallas as pl
from jax.experimental.pallas import tpu as pltpu
from jax.experimental.pallas import tpu_sc as plsc

_N = 10000
_E = 320000
_D = 128

_NC = 2
_NS = 16
_NW = _NC * _NS

_CHUNK = 125
_G = 5
_CPG = 16
_NCHUNK = _G * _CPG
_EPT = _NCHUNK * _CHUNK
_NPAD = 10112
_RPT = _NPAD // _NS


def _sc_aggregate_kernel(h_hbm, edge_hbm, z_hbm, out_hbm,
                         agg, src_blk, dst_blk, rows0, rows1,
                         sem0, sem1, sem_isrc, sem_idst):
    c = lax.axis_index("c")
    s = lax.axis_index("s")
    wid = c * _NS + s

    def start_idx(g):
        slot = g % 2
        pltpu.async_copy(edge_hbm.at[0, wid, g], src_blk.at[slot], sem_isrc)
        pltpu.async_copy(edge_hbm.at[1, wid, g], dst_blk.at[slot], sem_idst)

    def wait_idx(g):
        slot = g % 2
        pltpu.make_async_copy(edge_hbm.at[0, wid, g], src_blk.at[slot],
                              sem_isrc).wait()
        pltpu.make_async_copy(edge_hbm.at[1, wid, g], dst_blk.at[slot],
                              sem_idst).wait()

    start_idx(0)

    pltpu.sync_copy(z_hbm.at[pl.ds(s * _RPT, _RPT)],
                    agg.at[pl.ds(s * _RPT, _RPT)])
    plsc.subcore_barrier()

    bufs = (rows0, rows1)
    sems = (sem0, sem1)

    def start_gather(slot, j, b):
        pltpu.async_copy(h_hbm.at[src_blk.at[slot, j]], bufs[b], sems[b])

    def wait_gather(slot, j, b):
        pltpu.make_async_copy(h_hbm.at[src_blk.at[slot, j]], bufs[b],
                              sems[b]).wait()

    for g in range(_G):
        slot = g % 2
        wait_idx(g)
        if g + 1 < _G:
            start_idx(g + 1)
        start_gather(slot, 0, 0)
        start_gather(slot, 1, 1)

        def body(jj, carry):
            for b in range(2):
                j = 2 * jj + b
                wait_gather(slot, j, b)
                pltpu.sync_copy(bufs[b], agg.at[dst_blk.at[slot, j]],
                                add=True)

                @pl.when(j + 2 < _CPG)
                def _():
                    start_gather(slot, j + 2, b)
            return carry

        lax.fori_loop(0, _CPG // 2, body, 0)

    plsc.subcore_barrier()

    pltpu.sync_copy(agg.at[pl.ds(s * _RPT, _RPT)],
                    out_hbm.at[c, pl.ds(s * _RPT, _RPT)])


def _sc_aggregate(h, edge5, zeros_n):
    mesh = plsc.VectorSubcoreMesh(core_axis_name="c", subcore_axis_name="s")
    return pl.kernel(
        _sc_aggregate_kernel,
        out_type=jax.ShapeDtypeStruct((_NC, _NPAD, _D), jnp.float32),
        mesh=mesh,
        scratch_types=[
            pltpu.VMEM_SHARED((_NPAD, _D), jnp.float32),
            pltpu.VMEM((2, _CPG, _CHUNK), jnp.int32),
            pltpu.VMEM((2, _CPG, _CHUNK), jnp.int32),
            pltpu.VMEM((_CHUNK, _D), jnp.float32),
            pltpu.VMEM((_CHUNK, _D), jnp.float32),
            pltpu.SemaphoreType.DMA,
            pltpu.SemaphoreType.DMA,
            pltpu.SemaphoreType.DMA,
            pltpu.SemaphoreType.DMA,
        ],
    )(h, edge5, zeros_n)


def _tc_root_kernel(h_ref, wo_ref, b_ref, o_ref):
    o_ref[...] = jnp.dot(h_ref[...], wo_ref[...],
                         preferred_element_type=jnp.float32) + b_ref[...]


def _tc_root(h, w_root_t, b_rel):
    blk = 2000
    return pl.pallas_call(
        _tc_root_kernel,
        grid=(_N // blk,),
        in_specs=[
            pl.BlockSpec((blk, _D), lambda i: (i, 0)),
            pl.BlockSpec((_D, _D), lambda i: (0, 0)),
            pl.BlockSpec((1, _D), lambda i: (0, 0)),
        ],
        out_specs=pl.BlockSpec((blk, _D), lambda i: (i, 0)),
        out_shape=jax.ShapeDtypeStruct((_N, _D), jnp.float32),
    )(h, w_root_t, b_rel)


def _tc_combine_kernel(do_relu, p_ref, r_ref, wr_ref, o_ref):
    agg = p_ref[0] + p_ref[1]
    acc = jnp.dot(agg, wr_ref[...], preferred_element_type=jnp.float32)
    acc = acc + r_ref[...]
    if do_relu:
        acc = jnp.maximum(acc, 0.0)
    o_ref[...] = acc


def _tc_combine(partials, root, w_rel_t, do_relu):
    blk = 2000
    return pl.pallas_call(
        functools.partial(_tc_combine_kernel, do_relu),
        grid=(_N // blk,),
        in_specs=[
            pl.BlockSpec((_NC, blk, _D), lambda i: (0, i, 0)),
            pl.BlockSpec((blk, _D), lambda i: (i, 0)),
            pl.BlockSpec((_D, _D), lambda i: (0, 0)),
        ],
        out_specs=pl.BlockSpec((blk, _D), lambda i: (i, 0)),
        out_shape=jax.ShapeDtypeStruct((_N, _D), jnp.float32),
    )(partials, root, w_rel_t)


def kernel(x, edge_index, W1_rel, b1_rel, W1_root, W2_rel, b2_rel, W2_root,
           W3_rel, b3_rel, W3_root):
    edge5 = edge_index.reshape(2, _NW, _G, _CPG, _CHUNK)
    zeros_n = jnp.zeros((_NPAD, _D), jnp.float32)

    h = x
    layers = (
        (W1_rel, b1_rel, W1_root, True),
        (W2_rel, b2_rel, W2_root, True),
        (W3_rel, b3_rel, W3_root, False),
    )
    for w_rel, b_rel, w_root, do_relu in layers:
        partials = _sc_aggregate(h, edge5, zeros_n)
        root = _tc_root(h, w_root.T, b_rel.reshape(1, _D))
        h = _tc_combine(partials, root, w_rel.T, do_relu)
    return h
